# baseline (speedup 1.0000x reference)
import jax
import jax.numpy as jnp
from jax.experimental import pallas as pl
from jax.experimental.pallas import tpu as pltpu

N = 10000
E = 320000
D = 128
NB = 16
C = E // NB


def _fused_kernel(x_ref, src_ref, dst_ref, w_ref, b_ref, a_ref, out_ref,
                  h_ref, et_ref, denom_ref, agg_ref):
    i = pl.program_id(0)

    @pl.when(i == 0)
    def _prologue():
        h = jax.lax.dot_general(
            x_ref[:], w_ref[:], (((1,), (1,)), ((), ())),
            preferred_element_type=jnp.float32) + b_ref[:]
        h_ref[:] = h
        z = jax.lax.dot_general(
            h, a_ref[:], (((1,), (0,)), ((), ())),
            preferred_element_type=jnp.float32)
        t = jnp.where(z >= 0, z, 0.2 * z)
        gmax = jnp.max(t)
        et_ref[:] = jnp.exp(t - gmax)
        denom_ref[:] = jnp.zeros_like(denom_ref)
        agg_ref[:] = jnp.full_like(agg_ref, jnp.inf)

    def body(e, carry):
        s = src_ref[0, 0, e]
        d = dst_ref[0, 0, e]
        ex = et_ref[pl.ds(s, 1), :]
        denom_ref[pl.ds(d, 1), :] = denom_ref[pl.ds(d, 1), :] + ex
        row = h_ref[pl.ds(s, 1), :] * ex
        agg_ref[pl.ds(d, 1), :] = jnp.minimum(agg_ref[pl.ds(d, 1), :], row)
        return carry

    jax.lax.fori_loop(0, C, body, 0, unroll=False)

    @pl.when(i == NB - 1)
    def _epilogue():
        agg = agg_ref[:]
        scaled = agg / (denom_ref[:] + 1e-16)
        out_ref[:] = jnp.where(jnp.isfinite(agg), scaled, 0.0) + x_ref[:]


def kernel(x, edge_index, W_msg, b_msg, att_msg):
    src = edge_index[0].reshape(NB, 1, C)
    dst = edge_index[1].reshape(NB, 1, C)
    b = b_msg.reshape(1, D)
    a = att_msg.reshape(D, 1)
    return pl.pallas_call(
        _fused_kernel,
        grid=(NB,),
        out_shape=jax.ShapeDtypeStruct((N, D), jnp.float32),
        in_specs=[
            pl.BlockSpec((N, D), lambda i: (0, 0)),
            pl.BlockSpec((1, 1, C), lambda i: (i, 0, 0),
                         memory_space=pltpu.SMEM),
            pl.BlockSpec((1, 1, C), lambda i: (i, 0, 0),
                         memory_space=pltpu.SMEM),
            pl.BlockSpec((D, D), lambda i: (0, 0)),
            pl.BlockSpec((1, D), lambda i: (0, 0)),
            pl.BlockSpec((D, 1), lambda i: (0, 0)),
        ],
        out_specs=pl.BlockSpec((N, D), lambda i: (0, 0)),
        scratch_shapes=[
            pltpu.VMEM((N, D), jnp.float32),
            pltpu.VMEM((N, 1), jnp.float32),
            pltpu.VMEM((N, 1), jnp.float32),
            pltpu.VMEM((N, D), jnp.float32),
        ],
    )(x, src, dst, W_msg, b, a)


# speedup vs baseline: 21.1175x; 21.1175x over previous
import jax
import jax.numpy as jnp
from jax.experimental import pallas as pl
from jax.experimental.pallas import tpu as pltpu

N = 10000
E = 320000
D = 128
NB = 16
C = E // NB


def _fused_kernel(x_ref, src_ref, dst_ref, w_ref, b_ref, a_ref, out_ref,
                  g_ref, et_ref, den0_ref, den1_ref, agg0_ref, agg1_ref):
    i = pl.program_id(0)

    @pl.when(i == 0)
    def _prologue():
        h = jax.lax.dot_general(
            x_ref[:], w_ref[:], (((1,), (1,)), ((), ())),
            preferred_element_type=jnp.float32) + b_ref[:]
        z = jax.lax.dot_general(
            h, a_ref[:], (((1,), (0,)), ((), ())),
            preferred_element_type=jnp.float32)
        t = jnp.where(z >= 0, z, 0.2 * z)
        gmax = jnp.max(t)
        et = jnp.exp(t - gmax)
        et_ref[:] = et
        g_ref[:] = h * et
        den0_ref[:] = jnp.zeros_like(den0_ref)
        den1_ref[:] = jnp.zeros_like(den1_ref)
        agg0_ref[:] = jnp.full_like(agg0_ref, jnp.inf)
        agg1_ref[:] = jnp.full_like(agg1_ref, jnp.inf)

    def body(e, carry):
        s0 = src_ref[0, 0, 2 * e]
        d0 = dst_ref[0, 0, 2 * e]
        s1 = src_ref[0, 0, 2 * e + 1]
        d1 = dst_ref[0, 0, 2 * e + 1]
        den0_ref[pl.ds(d0, 1), :] = (den0_ref[pl.ds(d0, 1), :]
                                     + et_ref[pl.ds(s0, 1), :])
        den1_ref[pl.ds(d1, 1), :] = (den1_ref[pl.ds(d1, 1), :]
                                     + et_ref[pl.ds(s1, 1), :])
        agg0_ref[pl.ds(d0, 1), :] = jnp.minimum(
            agg0_ref[pl.ds(d0, 1), :], g_ref[pl.ds(s0, 1), :])
        agg1_ref[pl.ds(d1, 1), :] = jnp.minimum(
            agg1_ref[pl.ds(d1, 1), :], g_ref[pl.ds(s1, 1), :])
        return carry

    jax.lax.fori_loop(0, C // 2, body, 0, unroll=4)

    @pl.when(i == NB - 1)
    def _epilogue():
        agg = jnp.minimum(agg0_ref[:], agg1_ref[:])
        denom = den0_ref[:] + den1_ref[:]
        scaled = agg / (denom + 1e-16)
        out_ref[:] = jnp.where(jnp.isfinite(agg), scaled, 0.0) + x_ref[:]


def kernel(x, edge_index, W_msg, b_msg, att_msg):
    src = edge_index[0].reshape(NB, 1, C)
    dst = edge_index[1].reshape(NB, 1, C)
    b = b_msg.reshape(1, D)
    a = att_msg.reshape(D, 1)
    return pl.pallas_call(
        _fused_kernel,
        grid=(NB,),
        out_shape=jax.ShapeDtypeStruct((N, D), jnp.float32),
        in_specs=[
            pl.BlockSpec((N, D), lambda i: (0, 0)),
            pl.BlockSpec((1, 1, C), lambda i: (i, 0, 0),
                         memory_space=pltpu.SMEM),
            pl.BlockSpec((1, 1, C), lambda i: (i, 0, 0),
                         memory_space=pltpu.SMEM),
            pl.BlockSpec((D, D), lambda i: (0, 0)),
            pl.BlockSpec((1, D), lambda i: (0, 0)),
            pl.BlockSpec((D, 1), lambda i: (0, 0)),
        ],
        out_specs=pl.BlockSpec((N, D), lambda i: (0, 0)),
        scratch_shapes=[
            pltpu.VMEM((N, D), jnp.float32),
            pltpu.VMEM((N, 1), jnp.float32),
            pltpu.VMEM((N, 1), jnp.float32),
            pltpu.VMEM((N, 1), jnp.float32),
            pltpu.VMEM((N, D), jnp.float32),
            pltpu.VMEM((N, D), jnp.float32),
        ],
    )(x, src, dst, W_msg, b, a)
